# baseline (speedup 1.0000x reference)
import functools

import jax
import jax.numpy as jnp
from jax import lax
from jax.experimental import pallas as pl
from jax.experimental.pallas import tpu as pltpu
from jax.experimental.pallas import tpu_sc as plsc

F = 26
B = 4096
V = 100000
D = 32
NC = 13
FD = F * D
NT = F * (V + 1)

NW = 32
ROWS = F * B
RPW = ROWS // NW
CH = 128
NCHUNK = RPW // CH


def _sc_gather(w2f, w1f, idx3):
    mesh = plsc.VectorSubcoreMesh(core_axis_name="c", subcore_axis_name="s")

    @functools.partial(
        pl.kernel,
        out_type=[
            jax.ShapeDtypeStruct((NW, RPW, D), jnp.float32),
            jax.ShapeDtypeStruct((NW, NCHUNK, CH), jnp.float32),
        ],
        mesh=mesh,
        compiler_params=pltpu.CompilerParams(use_tc_tiling_on_sc=False),
        scratch_types=[
            pltpu.VMEM((NCHUNK, CH), jnp.int32),
            pltpu.VMEM((RPW, D), jnp.float32),
            pltpu.VMEM((NCHUNK, CH), jnp.float32),
            pltpu.SemaphoreType.DMA,
            pltpu.SemaphoreType.DMA,
        ],
    )
    def k(w2_hbm, w1_hbm, idx_hbm, emb_out, fo1_out, idx_v, emb_v, fo1_v, s2, s1):
        wid = lax.axis_index("s") * 2 + lax.axis_index("c")
        pltpu.sync_copy(idx_hbm.at[wid], idx_v)

        def fire(i, carry):
            pltpu.async_copy(w2_hbm.at[idx_v.at[i]], emb_v.at[pl.ds(i * CH, CH)], s2)
            pltpu.async_copy(w1_hbm.at[idx_v.at[i]], fo1_v.at[i], s1)
            return carry

        lax.fori_loop(0, NCHUNK, fire, 0)
        pltpu.make_async_copy(w2_hbm.at[pl.ds(0, RPW)], emb_v, s2).wait()

        def drain1(i, carry):
            pltpu.make_async_copy(w1_hbm.at[pl.ds(0, CH)], fo1_v.at[0], s1).wait()
            return carry

        lax.fori_loop(0, NCHUNK, drain1, 0)

        pltpu.sync_copy(emb_v, emb_out.at[wid])
        pltpu.sync_copy(fo1_v, fo1_out.at[wid])

    return k(w2f, w1f, idx3)


def _dense_body(emb_ref, fo1_ref, num_ref, fm1w_ref, fm1b_ref,
                w0a_ref, w0b_ref, b0_ref, g0_ref, be0_ref,
                w1_ref, b1_ref, g1_ref, be1_ref,
                w2_ref, b2_ref, g2_ref, be2_ref,
                ow_ref, ob_ref, out_ref):
    emb = emb_ref[...]
    num = num_ref[...]
    fo1 = fo1_ref[...]
    fm1w = fm1w_ref[...]

    fm1 = (jnp.sum(fo1 * fm1w[:, :F], axis=1, keepdims=True)
           + jnp.sum(num * fm1w[:, F:], axis=1, keepdims=True)
           + fm1b_ref[...])

    r = lax.broadcasted_iota(jnp.int32, (FD, D), 0)
    c = lax.broadcasted_iota(jnp.int32, (FD, D), 1)
    sel = jnp.where(r % D == c, 1.0, 0.0).astype(jnp.float32)
    sum_e = jnp.dot(emb, sel, preferred_element_type=jnp.float32)
    sum_sq = jnp.sum(sum_e * sum_e, axis=1, keepdims=True)
    sq_sum = jnp.sum(emb * emb, axis=1, keepdims=True)
    fm2 = 0.5 * (sum_sq - sq_sum)

    def layer(x_list, w_list, bias, g, be):
        h = bias
        for x, w in zip(x_list, w_list):
            h = h + jnp.dot(x, w, preferred_element_type=jnp.float32)
        mu = jnp.mean(h, axis=0, keepdims=True)
        var = jnp.mean((h - mu) * (h - mu), axis=0, keepdims=True)
        h = (h - mu) * lax.rsqrt(var + 1e-5) * g + be
        return jnp.maximum(h, 0.0)

    h0 = layer([emb, num], [w0a_ref[...], w0b_ref[...]], b0_ref[...], g0_ref[...], be0_ref[...])
    h1 = layer([h0], [w1_ref[...]], b1_ref[...], g1_ref[...], be1_ref[...])
    h2 = layer([h1], [w2_ref[...]], b2_ref[...], g2_ref[...], be2_ref[...])

    deep = jnp.dot(h2, ow_ref[...], preferred_element_type=jnp.float32) + ob_ref[...]
    out_ref[...] = jax.nn.sigmoid(fm1 + fm2 + deep)


def _dense(emb, fo1, num_fea, fm1_W, fm1_b,
           DW0, Db0, G0, Be0, DW1, Db1, G1, Be1, DW2, Db2, G2, Be2, OW, Ob):
    row2 = lambda v: v.reshape(1, -1)
    return pl.pallas_call(
        _dense_body,
        out_shape=jax.ShapeDtypeStruct((B, 1), jnp.float32),
        compiler_params=pltpu.CompilerParams(vmem_limit_bytes=120 * 1024 * 1024),
    )(emb, fo1, num_fea, fm1_W, row2(fm1_b),
      DW0[:, :FD].T, DW0[:, FD:].T, row2(Db0), row2(G0), row2(Be0),
      DW1.T, row2(Db1), row2(G1), row2(Be1),
      DW2.T, row2(Db2), row2(G2), row2(Be2),
      OW.T, row2(Ob))


def kernel(cat_fea, num_fea, W1, W2, fm1_W, fm1_b,
           DW0, Db0, G0, Be0, DW1, Db1, G1, Be1, DW2, Db2, G2, Be2, OW, Ob):
    w2f = W2.reshape(NT, D)
    w1f = W1.reshape(NT)
    offs = (jnp.arange(F, dtype=jnp.int32) * (V + 1))[:, None]
    idx3 = (cat_fea.astype(jnp.int32) + offs).T.reshape(NW, NCHUNK, CH)

    emb_w, fo1_w = _sc_gather(w2f, w1f, idx3)
    emb = emb_w.reshape(B, FD)
    fo1 = fo1_w.reshape(B, F)

    return _dense(emb, fo1, num_fea, fm1_W, fm1_b,
                  DW0, Db0, G0, Be0, DW1, Db1, G1, Be1,
                  DW2, Db2, G2, Be2, OW, Ob)


# speedup vs baseline: 8.7848x; 8.7848x over previous
import functools

import jax
import jax.numpy as jnp
from jax import lax
from jax.experimental import pallas as pl
from jax.experimental.pallas import tpu as pltpu
from jax.experimental.pallas import tpu_sc as plsc

F = 26
B = 4096
V = 100000
D = 32
NC = 13
FD = F * D
NT1 = F * (V + 1)
VB = 2048
NQ = 49
VP = NQ * VB
NT2 = F * VP

NW = 32
ROWS = F * B
RPW = ROWS // NW
CH = 128
NCHUNK = RPW // CH


def _linearize_body(in_ref, out_ref):
    x = in_ref[0]
    xt = jnp.transpose(x)
    xt3 = xt.reshape(VB // 4, 4, D)
    out_ref[...] = jnp.concatenate([xt3[:, j, :] for j in range(4)], axis=1)


def _linearize(w2t):
    return pl.pallas_call(
        _linearize_body,
        grid=(F, NQ),
        in_specs=[pl.BlockSpec((1, D, VB), lambda f, q: (f, 0, q))],
        out_specs=pl.BlockSpec((VB * D // 128, 128), lambda f, q: (f * NQ + q, 0)),
        out_shape=jax.ShapeDtypeStruct((NT2 * D // 128, 128), jnp.float32),
    )(w2t)


def _sc_gather(w2f, w1f, idx2, idx1):
    mesh = plsc.VectorSubcoreMesh(core_axis_name="c", subcore_axis_name="s")

    @functools.partial(
        pl.kernel,
        out_type=[
            jax.ShapeDtypeStruct((NW, RPW, D), jnp.float32),
            jax.ShapeDtypeStruct((NW, NCHUNK, CH), jnp.float32),
        ],
        mesh=mesh,
        compiler_params=pltpu.CompilerParams(use_tc_tiling_on_sc=False),
        scratch_types=[
            pltpu.VMEM((NCHUNK, CH), jnp.int32),
            pltpu.VMEM((NCHUNK, CH), jnp.int32),
            pltpu.VMEM((RPW, D), jnp.float32),
            pltpu.VMEM((NCHUNK, CH), jnp.float32),
            pltpu.SemaphoreType.DMA,
            pltpu.SemaphoreType.DMA,
        ],
    )
    def k(w2_hbm, w1_hbm, i2_hbm, i1_hbm, emb_out, fo1_out,
          i2_v, i1_v, emb_v, fo1_v, s2, s1):
        wid = lax.axis_index("s") * 2 + lax.axis_index("c")
        pltpu.sync_copy(i2_hbm.at[wid], i2_v)
        pltpu.sync_copy(i1_hbm.at[wid], i1_v)

        def fire(i, carry):
            pltpu.async_copy(w2_hbm.at[i2_v.at[i]], emb_v.at[pl.ds(i * CH, CH)], s2)
            pltpu.async_copy(w1_hbm.at[i1_v.at[i]], fo1_v.at[i], s1)
            return carry

        lax.fori_loop(0, NCHUNK, fire, 0)
        pltpu.make_async_copy(w2_hbm.at[pl.ds(0, RPW)], emb_v, s2).wait()

        def drain1(i, carry):
            pltpu.make_async_copy(w1_hbm.at[pl.ds(0, CH)], fo1_v.at[0], s1).wait()
            return carry

        lax.fori_loop(0, NCHUNK, drain1, 0)

        pltpu.sync_copy(emb_v, emb_out.at[wid])
        pltpu.sync_copy(fo1_v, fo1_out.at[wid])

    return k(w2f, w1f, idx2, idx1)


def _dense_body(emb_ref, fo1_ref, num_ref, fm1w_ref, fm1b_ref,
                w0a_ref, w0b_ref, b0_ref, g0_ref, be0_ref,
                w1_ref, b1_ref, g1_ref, be1_ref,
                w2_ref, b2_ref, g2_ref, be2_ref,
                ow_ref, ob_ref, out_ref):
    emb = emb_ref[...]
    num = num_ref[...]
    fo1 = fo1_ref[...]
    fm1w = fm1w_ref[...]

    fm1 = (jnp.sum(fo1 * fm1w[:, :F], axis=1, keepdims=True)
           + jnp.sum(num * fm1w[:, F:], axis=1, keepdims=True)
           + fm1b_ref[...])

    r = lax.broadcasted_iota(jnp.int32, (FD, D), 0)
    c = lax.broadcasted_iota(jnp.int32, (FD, D), 1)
    sel = jnp.where(r % D == c, 1.0, 0.0).astype(jnp.float32)
    sum_e = jnp.dot(emb, sel, preferred_element_type=jnp.float32)
    sum_sq = jnp.sum(sum_e * sum_e, axis=1, keepdims=True)
    sq_sum = jnp.sum(emb * emb, axis=1, keepdims=True)
    fm2 = 0.5 * (sum_sq - sq_sum)

    def layer(x_list, w_list, bias, g, be):
        h = bias
        for x, w in zip(x_list, w_list):
            h = h + jnp.dot(x, w, preferred_element_type=jnp.float32)
        mu = jnp.mean(h, axis=0, keepdims=True)
        var = jnp.mean((h - mu) * (h - mu), axis=0, keepdims=True)
        h = (h - mu) * lax.rsqrt(var + 1e-5) * g + be
        return jnp.maximum(h, 0.0)

    h0 = layer([emb, num], [w0a_ref[...], w0b_ref[...]], b0_ref[...], g0_ref[...], be0_ref[...])
    h1 = layer([h0], [w1_ref[...]], b1_ref[...], g1_ref[...], be1_ref[...])
    h2 = layer([h1], [w2_ref[...]], b2_ref[...], g2_ref[...], be2_ref[...])

    deep = jnp.dot(h2, ow_ref[...], preferred_element_type=jnp.float32) + ob_ref[...]
    out_ref[...] = jax.nn.sigmoid(fm1 + fm2 + deep)


def _dense(emb, fo1, num_fea, fm1_W, fm1_b,
           DW0, Db0, G0, Be0, DW1, Db1, G1, Be1, DW2, Db2, G2, Be2, OW, Ob):
    row2 = lambda v: v.reshape(1, -1)
    return pl.pallas_call(
        _dense_body,
        out_shape=jax.ShapeDtypeStruct((B, 1), jnp.float32),
        compiler_params=pltpu.CompilerParams(vmem_limit_bytes=120 * 1024 * 1024),
    )(emb, fo1, num_fea, fm1_W, row2(fm1_b),
      DW0[:, :FD].T, DW0[:, FD:].T, row2(Db0), row2(G0), row2(Be0),
      DW1.T, row2(Db1), row2(G1), row2(Be1),
      DW2.T, row2(Db2), row2(G2), row2(Be2),
      OW.T, row2(Ob))


def kernel(cat_fea, num_fea, W1, W2, fm1_W, fm1_b,
           DW0, Db0, G0, Be0, DW1, Db1, G1, Be1, DW2, Db2, G2, Be2, OW, Ob):
    w2t = jnp.transpose(W2, (0, 2, 1))
    w2f = _linearize(w2t).reshape(NT2, D)
    w1f = W1.reshape(NT1)
    ci = cat_fea.astype(jnp.int32)
    offs2 = (jnp.arange(F, dtype=jnp.int32) * VP)[:, None]
    offs1 = (jnp.arange(F, dtype=jnp.int32) * (V + 1))[:, None]
    idx2 = (ci + offs2).T.reshape(NW, NCHUNK, CH)
    idx1 = (ci + offs1).T.reshape(NW, NCHUNK, CH)

    emb_w, fo1_w = _sc_gather(w2f, w1f, idx2, idx1)
    emb = emb_w.reshape(B, FD)
    fo1 = fo1_w.reshape(B, F)

    return _dense(emb, fo1, num_fea, fm1_W, fm1_b,
                  DW0, Db0, G0, Be0, DW1, Db1, G1, Be1,
                  DW2, Db2, G2, Be2, OW, Ob)
